# baseline (device time: 312686 ns/iter reference)
import jax
import jax.numpy as jnp
from jax import lax
from jax.experimental import pallas as pl
from jax.experimental.pallas import tpu as pltpu

N_DEV = 4


def _gelu(y):
    c = 0.7978845608028654
    return 0.5 * y * (1.0 + jnp.tanh(c * (y + 0.044715 * y * y * y)))


def kernel(x, w_mat):
    m, k_per = x.shape
    _, n = w_mat.shape
    chunk = m // N_DEV

    def body(x_ref, w_ref, out_ref, comm_ref, rs_send, rs_recv, ag_send, ag_recv):
        my = lax.axis_index("i")
        left = lax.rem(my + N_DEV - 1, N_DEV)
        right = lax.rem(my + 1, N_DEV)

        barrier = pltpu.get_barrier_semaphore()
        for nbr in (left, right):
            pl.semaphore_signal(
                barrier, inc=1,
                device_id=(nbr,), device_id_type=pl.DeviceIdType.MESH,
            )
        pl.semaphore_wait(barrier, 2)

        out_ref[:, :] = jnp.dot(
            x_ref[:, :], w_ref[:, :], preferred_element_type=jnp.float32
        )

        def rows(c):
            return pl.ds(c * chunk, chunk)

        for h in range(N_DEV - 1):
            c_send = lax.rem(my - h + N_DEV, N_DEV)
            c_recv = lax.rem(my - h - 1 + N_DEV, N_DEV)
            rdma = pltpu.make_async_remote_copy(
                src_ref=out_ref.at[rows(c_send)],
                dst_ref=comm_ref.at[h],
                send_sem=rs_send.at[h],
                recv_sem=rs_recv.at[h],
                device_id=(right,),
                device_id_type=pl.DeviceIdType.MESH,
            )
            rdma.start()
            rdma.wait()
            out_ref[rows(c_recv), :] = out_ref[rows(c_recv), :] + comm_ref[h, :, :]

        c_mine = lax.rem(my + 1, N_DEV)
        out_ref[rows(c_mine), :] = _gelu(out_ref[rows(c_mine), :])

        for h in range(N_DEV - 1):
            c_send = lax.rem(my + 1 - h + N_DEV, N_DEV)
            rdma = pltpu.make_async_remote_copy(
                src_ref=out_ref.at[rows(c_send)],
                dst_ref=out_ref.at[rows(c_send)],
                send_sem=ag_send.at[h],
                recv_sem=ag_recv.at[h],
                device_id=(right,),
                device_id_type=pl.DeviceIdType.MESH,
            )
            rdma.start()
            rdma.wait()

    return pl.pallas_call(
        body,
        out_shape=jax.ShapeDtypeStruct((m, n), jnp.float32),
        in_specs=[
            pl.BlockSpec(memory_space=pltpu.VMEM),
            pl.BlockSpec(memory_space=pltpu.VMEM),
        ],
        out_specs=pl.BlockSpec(memory_space=pltpu.VMEM),
        scratch_shapes=[
            pltpu.VMEM((N_DEV - 1, chunk, n), jnp.float32),
            pltpu.SemaphoreType.DMA((N_DEV - 1,)),
            pltpu.SemaphoreType.DMA((N_DEV - 1,)),
            pltpu.SemaphoreType.DMA((N_DEV - 1,)),
            pltpu.SemaphoreType.DMA((N_DEV - 1,)),
        ],
        compiler_params=pltpu.CompilerParams(collective_id=0),
    )(x, w_mat)


# device time: 178189 ns/iter; 1.7548x vs baseline; 1.7548x over previous
import jax
import jax.numpy as jnp
from jax import lax
from jax.experimental import pallas as pl
from jax.experimental.pallas import tpu as pltpu

N_DEV = 4


def _gelu(y):
    c = 0.7978845608028654
    return 0.5 * y * (1.0 + jnp.tanh(c * (y + 0.044715 * y * y * y)))


def kernel(x, w_mat):
    m, k_per = x.shape
    _, n = w_mat.shape
    chunk = m // N_DEV
    half = n // 2

    def body(
        x_ref, w_ref, out_ref,
        comm_r, comm_l,
        rs_send_r, rs_recv_r, rs_send_l, rs_recv_l,
        ag_send_r, ag_recv_r, ag_send_l, ag_recv_l,
    ):
        my = lax.axis_index("i")
        left = lax.rem(my + N_DEV - 1, N_DEV)
        right = lax.rem(my + 1, N_DEV)

        barrier = pltpu.get_barrier_semaphore()
        for nbr in (left, right):
            pl.semaphore_signal(
                barrier, inc=1,
                device_id=(nbr,), device_id_type=pl.DeviceIdType.MESH,
            )
        pl.semaphore_wait(barrier, 2)

        out_ref[:, :] = jnp.dot(
            x_ref[:, :], w_ref[:, :], preferred_element_type=jnp.float32
        )

        def rows(c):
            return pl.ds(c * chunk, chunk)

        cols_a = pl.ds(0, half)
        cols_b = pl.ds(half, half)

        for h in range(N_DEV - 1):
            cs_r = lax.rem(my - h + N_DEV, N_DEV)
            cr_r = lax.rem(my - h - 1 + N_DEV, N_DEV)
            cs_l = lax.rem(my + h, N_DEV)
            cr_l = lax.rem(my + h + 1, N_DEV)
            rdma_r = pltpu.make_async_remote_copy(
                src_ref=out_ref.at[rows(cs_r), cols_a],
                dst_ref=comm_r.at[h],
                send_sem=rs_send_r.at[h],
                recv_sem=rs_recv_r.at[h],
                device_id=(right,),
                device_id_type=pl.DeviceIdType.MESH,
            )
            rdma_l = pltpu.make_async_remote_copy(
                src_ref=out_ref.at[rows(cs_l), cols_b],
                dst_ref=comm_l.at[h],
                send_sem=rs_send_l.at[h],
                recv_sem=rs_recv_l.at[h],
                device_id=(left,),
                device_id_type=pl.DeviceIdType.MESH,
            )
            rdma_r.start()
            rdma_l.start()
            rdma_r.wait()
            rdma_l.wait()
            out_ref[rows(cr_r), cols_a] = (
                out_ref[rows(cr_r), cols_a] + comm_r[h, :, :]
            )
            out_ref[rows(cr_l), cols_b] = (
                out_ref[rows(cr_l), cols_b] + comm_l[h, :, :]
            )

        c_own_r = lax.rem(my + 1, N_DEV)
        c_own_l = lax.rem(my + N_DEV - 1, N_DEV)
        out_ref[rows(c_own_r), cols_a] = _gelu(out_ref[rows(c_own_r), cols_a])
        out_ref[rows(c_own_l), cols_b] = _gelu(out_ref[rows(c_own_l), cols_b])

        for h in range(N_DEV - 1):
            cs_r = lax.rem(my + 1 - h + N_DEV, N_DEV)
            cs_l = lax.rem(my - 1 + h + N_DEV, N_DEV)
            rdma_r = pltpu.make_async_remote_copy(
                src_ref=out_ref.at[rows(cs_r), cols_a],
                dst_ref=out_ref.at[rows(cs_r), cols_a],
                send_sem=ag_send_r.at[h],
                recv_sem=ag_recv_r.at[h],
                device_id=(right,),
                device_id_type=pl.DeviceIdType.MESH,
            )
            rdma_l = pltpu.make_async_remote_copy(
                src_ref=out_ref.at[rows(cs_l), cols_b],
                dst_ref=out_ref.at[rows(cs_l), cols_b],
                send_sem=ag_send_l.at[h],
                recv_sem=ag_recv_l.at[h],
                device_id=(left,),
                device_id_type=pl.DeviceIdType.MESH,
            )
            rdma_r.start()
            rdma_l.start()
            rdma_r.wait()
            rdma_l.wait()

    return pl.pallas_call(
        body,
        out_shape=jax.ShapeDtypeStruct((m, n), jnp.float32),
        in_specs=[
            pl.BlockSpec(memory_space=pltpu.VMEM),
            pl.BlockSpec(memory_space=pltpu.VMEM),
        ],
        out_specs=pl.BlockSpec(memory_space=pltpu.VMEM),
        scratch_shapes=[
            pltpu.VMEM((N_DEV - 1, chunk, half), jnp.float32),
            pltpu.VMEM((N_DEV - 1, chunk, half), jnp.float32),
            pltpu.SemaphoreType.DMA((N_DEV - 1,)),
            pltpu.SemaphoreType.DMA((N_DEV - 1,)),
            pltpu.SemaphoreType.DMA((N_DEV - 1,)),
            pltpu.SemaphoreType.DMA((N_DEV - 1,)),
            pltpu.SemaphoreType.DMA((N_DEV - 1,)),
            pltpu.SemaphoreType.DMA((N_DEV - 1,)),
            pltpu.SemaphoreType.DMA((N_DEV - 1,)),
            pltpu.SemaphoreType.DMA((N_DEV - 1,)),
        ],
        compiler_params=pltpu.CompilerParams(collective_id=0),
    )(x, w_mat)


# device time: 174367 ns/iter; 1.7933x vs baseline; 1.0219x over previous
import jax
import jax.numpy as jnp
from jax import lax
from jax.experimental import pallas as pl
from jax.experimental.pallas import tpu as pltpu

N_DEV = 4


def _gelu(y):
    c = 0.7978845608028654
    return 0.5 * y * (1.0 + jnp.tanh(c * (y + 0.044715 * y * y * y)))


def kernel(x, w_mat):
    m, k_per = x.shape
    _, n = w_mat.shape
    chunk = m // N_DEV
    half = n // 2

    def body(
        x_ref, w_ref, out_ref,
        comm_r, comm_l,
        rs_send_r, rs_recv_r, rs_send_l, rs_recv_l,
        ag_send_r, ag_recv_r, ag_send_l, ag_recv_l,
    ):
        my = lax.axis_index("i")
        left = lax.rem(my + N_DEV - 1, N_DEV)
        right = lax.rem(my + 1, N_DEV)

        barrier = pltpu.get_barrier_semaphore()
        for nbr in (left, right):
            pl.semaphore_signal(
                barrier, inc=1,
                device_id=(nbr,), device_id_type=pl.DeviceIdType.MESH,
            )
        pl.semaphore_wait(barrier, 2)

        def rows(c):
            return pl.ds(c * chunk, chunk)

        cols_a = pl.ds(0, half)
        cols_b = pl.ds(half, half)

        def gemm_chunk(c):
            out_ref[rows(c), :] = jnp.dot(
                x_ref[rows(c), :], w_ref[:, :],
                preferred_element_type=jnp.float32,
            )

        gemm_chunk(my)

        for h in range(N_DEV - 1):
            cs_r = lax.rem(my - h + N_DEV, N_DEV)
            cr_r = lax.rem(my - h - 1 + N_DEV, N_DEV)
            cs_l = lax.rem(my + h, N_DEV)
            cr_l = lax.rem(my + h + 1, N_DEV)
            rdma_r = pltpu.make_async_remote_copy(
                src_ref=out_ref.at[rows(cs_r), cols_a],
                dst_ref=comm_r.at[h],
                send_sem=rs_send_r.at[h],
                recv_sem=rs_recv_r.at[h],
                device_id=(right,),
                device_id_type=pl.DeviceIdType.MESH,
            )
            rdma_l = pltpu.make_async_remote_copy(
                src_ref=out_ref.at[rows(cs_l), cols_b],
                dst_ref=comm_l.at[h],
                send_sem=rs_send_l.at[h],
                recv_sem=rs_recv_l.at[h],
                device_id=(left,),
                device_id_type=pl.DeviceIdType.MESH,
            )
            rdma_r.start()
            rdma_l.start()
            if h == 0:
                for j in range(1, N_DEV):
                    gemm_chunk(lax.rem(my + j, N_DEV))
            rdma_r.wait()
            rdma_l.wait()
            if h == N_DEV - 2:
                out_ref[rows(cr_r), cols_a] = _gelu(
                    out_ref[rows(cr_r), cols_a] + comm_r[h, :, :]
                )
                out_ref[rows(cr_l), cols_b] = _gelu(
                    out_ref[rows(cr_l), cols_b] + comm_l[h, :, :]
                )
            else:
                out_ref[rows(cr_r), cols_a] = (
                    out_ref[rows(cr_r), cols_a] + comm_r[h, :, :]
                )
                out_ref[rows(cr_l), cols_b] = (
                    out_ref[rows(cr_l), cols_b] + comm_l[h, :, :]
                )

        for h in range(N_DEV - 1):
            cs_r = lax.rem(my + 1 - h + N_DEV, N_DEV)
            cs_l = lax.rem(my - 1 + h + N_DEV, N_DEV)
            rdma_r = pltpu.make_async_remote_copy(
                src_ref=out_ref.at[rows(cs_r), cols_a],
                dst_ref=out_ref.at[rows(cs_r), cols_a],
                send_sem=ag_send_r.at[h],
                recv_sem=ag_recv_r.at[h],
                device_id=(right,),
                device_id_type=pl.DeviceIdType.MESH,
            )
            rdma_l = pltpu.make_async_remote_copy(
                src_ref=out_ref.at[rows(cs_l), cols_b],
                dst_ref=out_ref.at[rows(cs_l), cols_b],
                send_sem=ag_send_l.at[h],
                recv_sem=ag_recv_l.at[h],
                device_id=(left,),
                device_id_type=pl.DeviceIdType.MESH,
            )
            rdma_r.start()
            rdma_l.start()
            rdma_r.wait()
            rdma_l.wait()

    return pl.pallas_call(
        body,
        out_shape=jax.ShapeDtypeStruct((m, n), jnp.float32),
        in_specs=[
            pl.BlockSpec(memory_space=pltpu.VMEM),
            pl.BlockSpec(memory_space=pltpu.VMEM),
        ],
        out_specs=pl.BlockSpec(memory_space=pltpu.VMEM),
        scratch_shapes=[
            pltpu.VMEM((N_DEV - 1, chunk, half), jnp.float32),
            pltpu.VMEM((N_DEV - 1, chunk, half), jnp.float32),
            pltpu.SemaphoreType.DMA((N_DEV - 1,)),
            pltpu.SemaphoreType.DMA((N_DEV - 1,)),
            pltpu.SemaphoreType.DMA((N_DEV - 1,)),
            pltpu.SemaphoreType.DMA((N_DEV - 1,)),
            pltpu.SemaphoreType.DMA((N_DEV - 1,)),
            pltpu.SemaphoreType.DMA((N_DEV - 1,)),
            pltpu.SemaphoreType.DMA((N_DEV - 1,)),
            pltpu.SemaphoreType.DMA((N_DEV - 1,)),
        ],
        compiler_params=pltpu.CompilerParams(collective_id=0),
    )(x, w_mat)


# device time: 174329 ns/iter; 1.7937x vs baseline; 1.0002x over previous
import jax
import jax.numpy as jnp
from jax import lax
from jax.experimental import pallas as pl
from jax.experimental.pallas import tpu as pltpu

N_DEV = 4


def _gelu(y):
    c = 0.7978845608028654
    return 0.5 * y * (1.0 + jnp.tanh(c * (y + 0.044715 * y * y * y)))


def kernel(x, w_mat):
    m, k_per = x.shape
    _, n = w_mat.shape
    chunk = m // N_DEV
    half = n // 2

    def body(
        x_ref, w_ref, out_ref,
        stage, comm_r, comm_l,
        rs_send_r, rs_recv_r, rs_send_l, rs_recv_l,
        ag_send_r, ag_recv_r, ag_send_l, ag_recv_l,
    ):
        my = lax.axis_index("i")
        left = lax.rem(my + N_DEV - 1, N_DEV)
        right = lax.rem(my + 1, N_DEV)

        barrier = pltpu.get_barrier_semaphore()
        for nbr in (left, right):
            pl.semaphore_signal(
                barrier, inc=1,
                device_id=(nbr,), device_id_type=pl.DeviceIdType.MESH,
            )
        pl.semaphore_wait(barrier, 2)

        def rows(c):
            return pl.ds(c * chunk, chunk)

        cols_a = pl.ds(0, half)
        cols_b = pl.ds(half, half)

        def gemm_chunk(c):
            out_ref[rows(c), :] = jnp.dot(
                x_ref[rows(c), :], w_ref[:, :],
                preferred_element_type=jnp.float32,
            )

        stage[:, :] = jnp.dot(
            x_ref[rows(my), :], w_ref[:, :], preferred_element_type=jnp.float32
        )

        for h in range(N_DEV - 1):
            cs_r = lax.rem(my - h + N_DEV, N_DEV)
            cr_r = lax.rem(my - h - 1 + N_DEV, N_DEV)
            cs_l = lax.rem(my + h, N_DEV)
            cr_l = lax.rem(my + h + 1, N_DEV)
            src_r = (
                stage.at[pl.ds(0, chunk), cols_a]
                if h == 0
                else out_ref.at[rows(cs_r), cols_a]
            )
            src_l = (
                stage.at[pl.ds(0, chunk), cols_b]
                if h == 0
                else out_ref.at[rows(cs_l), cols_b]
            )
            rdma_r = pltpu.make_async_remote_copy(
                src_ref=src_r,
                dst_ref=comm_r.at[h],
                send_sem=rs_send_r.at[h],
                recv_sem=rs_recv_r.at[h],
                device_id=(right,),
                device_id_type=pl.DeviceIdType.MESH,
            )
            rdma_l = pltpu.make_async_remote_copy(
                src_ref=src_l,
                dst_ref=comm_l.at[h],
                send_sem=rs_send_l.at[h],
                recv_sem=rs_recv_l.at[h],
                device_id=(left,),
                device_id_type=pl.DeviceIdType.MESH,
            )
            rdma_r.start()
            rdma_l.start()
            if h == 0:
                for j in range(1, N_DEV):
                    gemm_chunk(lax.rem(my + j, N_DEV))
            rdma_r.wait()
            rdma_l.wait()
            if h == N_DEV - 2:
                out_ref[rows(cr_r), cols_a] = _gelu(
                    out_ref[rows(cr_r), cols_a] + comm_r[h, :, :]
                )
                out_ref[rows(cr_l), cols_b] = _gelu(
                    out_ref[rows(cr_l), cols_b] + comm_l[h, :, :]
                )
            else:
                out_ref[rows(cr_r), cols_a] = (
                    out_ref[rows(cr_r), cols_a] + comm_r[h, :, :]
                )
                out_ref[rows(cr_l), cols_b] = (
                    out_ref[rows(cr_l), cols_b] + comm_l[h, :, :]
                )

        for h in range(N_DEV - 1):
            cs_r = lax.rem(my + 1 - h + N_DEV, N_DEV)
            cs_l = lax.rem(my - 1 + h + N_DEV, N_DEV)
            rdma_r = pltpu.make_async_remote_copy(
                src_ref=out_ref.at[rows(cs_r), cols_a],
                dst_ref=out_ref.at[rows(cs_r), cols_a],
                send_sem=ag_send_r.at[h],
                recv_sem=ag_recv_r.at[h],
                device_id=(right,),
                device_id_type=pl.DeviceIdType.MESH,
            )
            rdma_l = pltpu.make_async_remote_copy(
                src_ref=out_ref.at[rows(cs_l), cols_b],
                dst_ref=out_ref.at[rows(cs_l), cols_b],
                send_sem=ag_send_l.at[h],
                recv_sem=ag_recv_l.at[h],
                device_id=(left,),
                device_id_type=pl.DeviceIdType.MESH,
            )
            rdma_r.start()
            rdma_l.start()
            rdma_r.wait()
            rdma_l.wait()

    return pl.pallas_call(
        body,
        out_shape=jax.ShapeDtypeStruct((m, n), jnp.float32),
        in_specs=[
            pl.BlockSpec(memory_space=pltpu.VMEM),
            pl.BlockSpec(memory_space=pltpu.VMEM),
        ],
        out_specs=pl.BlockSpec(memory_space=pltpu.VMEM),
        scratch_shapes=[
            pltpu.VMEM((chunk, n), jnp.float32),
            pltpu.VMEM((N_DEV - 1, chunk, half), jnp.float32),
            pltpu.VMEM((N_DEV - 1, chunk, half), jnp.float32),
            pltpu.SemaphoreType.DMA((N_DEV - 1,)),
            pltpu.SemaphoreType.DMA((N_DEV - 1,)),
            pltpu.SemaphoreType.DMA((N_DEV - 1,)),
            pltpu.SemaphoreType.DMA((N_DEV - 1,)),
            pltpu.SemaphoreType.DMA((N_DEV - 1,)),
            pltpu.SemaphoreType.DMA((N_DEV - 1,)),
            pltpu.SemaphoreType.DMA((N_DEV - 1,)),
            pltpu.SemaphoreType.DMA((N_DEV - 1,)),
        ],
        compiler_params=pltpu.CompilerParams(collective_id=0),
    )(x, w_mat)


# device time: 163162 ns/iter; 1.9164x vs baseline; 1.0684x over previous
import jax
import jax.numpy as jnp
from jax import lax
from jax.experimental import pallas as pl
from jax.experimental.pallas import tpu as pltpu

N_DEV = 4
S = 2


def _gelu(y):
    c = 0.7978845608028654
    return 0.5 * y * (1.0 + jnp.tanh(c * (y + 0.044715 * y * y * y)))


def kernel(x, w_mat):
    m, k_per = x.shape
    _, n = w_mat.shape
    chunk = m // N_DEV
    half = n // 2
    q = half // S

    def body(
        x_ref, w_ref, out_ref,
        stage, comm_r, comm_l,
        rs_send_r, rs_recv_r, rs_send_l, rs_recv_l,
        ag_send_r, ag_recv_r, ag_send_l, ag_recv_l,
    ):
        my = lax.axis_index("i")
        left = lax.rem(my + N_DEV - 1, N_DEV)
        right = lax.rem(my + 1, N_DEV)

        barrier = pltpu.get_barrier_semaphore()
        for nbr in (left, right):
            pl.semaphore_signal(
                barrier, inc=1,
                device_id=(nbr,), device_id_type=pl.DeviceIdType.MESH,
            )
        pl.semaphore_wait(barrier, 2)

        def rows(c):
            return pl.ds(c * chunk, chunk)

        def cols_r(s):
            return pl.ds(s * q, q)

        def cols_l(s):
            return pl.ds(half + s * q, q)

        def cs_r(h):
            return lax.rem(my - h + N_DEV, N_DEV)

        def cr_r(h):
            return lax.rem(my - h - 1 + N_DEV, N_DEV)

        def cs_l(h):
            return lax.rem(my + h, N_DEV)

        def cr_l(h):
            return lax.rem(my + h + 1, N_DEV)

        def make_rs(h, s):
            if h == 0:
                src_r = stage.at[pl.ds(0, chunk), cols_r(s)]
                src_l = stage.at[pl.ds(0, chunk), cols_l(s)]
            else:
                src_r = out_ref.at[rows(cs_r(h)), cols_r(s)]
                src_l = out_ref.at[rows(cs_l(h)), cols_l(s)]
            rdma_r = pltpu.make_async_remote_copy(
                src_ref=src_r,
                dst_ref=comm_r.at[h, pl.ds(0, chunk), pl.ds(s * q, q)],
                send_sem=rs_send_r.at[h, s],
                recv_sem=rs_recv_r.at[h, s],
                device_id=(right,),
                device_id_type=pl.DeviceIdType.MESH,
            )
            rdma_l = pltpu.make_async_remote_copy(
                src_ref=src_l,
                dst_ref=comm_l.at[h, pl.ds(0, chunk), pl.ds(s * q, q)],
                send_sem=rs_send_l.at[h, s],
                recv_sem=rs_recv_l.at[h, s],
                device_id=(left,),
                device_id_type=pl.DeviceIdType.MESH,
            )
            return rdma_r, rdma_l

        def make_ag(h, s):
            ca = lax.rem(my + 1 - h + N_DEV, N_DEV)
            cb = lax.rem(my - 1 + h + N_DEV, N_DEV)
            rdma_r = pltpu.make_async_remote_copy(
                src_ref=out_ref.at[rows(ca), cols_r(s)],
                dst_ref=out_ref.at[rows(ca), cols_r(s)],
                send_sem=ag_send_r.at[h, s],
                recv_sem=ag_recv_r.at[h, s],
                device_id=(right,),
                device_id_type=pl.DeviceIdType.MESH,
            )
            rdma_l = pltpu.make_async_remote_copy(
                src_ref=out_ref.at[rows(cb), cols_l(s)],
                dst_ref=out_ref.at[rows(cb), cols_l(s)],
                send_sem=ag_send_l.at[h, s],
                recv_sem=ag_recv_l.at[h, s],
                device_id=(left,),
                device_id_type=pl.DeviceIdType.MESH,
            )
            return rdma_r, rdma_l

        stage[:, :] = jnp.dot(
            x_ref[rows(my), :], w_ref[:, :], preferred_element_type=jnp.float32
        )

        rs = [[None] * S for _ in range(N_DEV - 1)]
        ag = [[None] * S for _ in range(N_DEV - 1)]

        for s in range(S):
            rs[0][s] = make_rs(0, s)
            rs[0][s][0].start()
            rs[0][s][1].start()

        for j in range(1, N_DEV):
            c = lax.rem(my + j, N_DEV)
            out_ref[rows(c), :] = jnp.dot(
                x_ref[rows(c), :], w_ref[:, :],
                preferred_element_type=jnp.float32,
            )

        for h in range(1, N_DEV - 1):
            for s in range(S):
                prev_r, prev_l = rs[h - 1][s]
                prev_r.wait_recv()
                prev_l.wait_recv()
                out_ref[rows(cr_r(h - 1)), cols_r(s)] = (
                    out_ref[rows(cr_r(h - 1)), cols_r(s)]
                    + comm_r[h - 1, :, pl.ds(s * q, q)]
                )
                out_ref[rows(cr_l(h - 1)), cols_l(s)] = (
                    out_ref[rows(cr_l(h - 1)), cols_l(s)]
                    + comm_l[h - 1, :, pl.ds(s * q, q)]
                )
                rs[h][s] = make_rs(h, s)
                rs[h][s][0].start()
                rs[h][s][1].start()

        hl = N_DEV - 2
        for s in range(S):
            prev_r, prev_l = rs[hl][s]
            prev_r.wait_recv()
            prev_l.wait_recv()
            out_ref[rows(cr_r(hl)), cols_r(s)] = _gelu(
                out_ref[rows(cr_r(hl)), cols_r(s)]
                + comm_r[hl, :, pl.ds(s * q, q)]
            )
            out_ref[rows(cr_l(hl)), cols_l(s)] = _gelu(
                out_ref[rows(cr_l(hl)), cols_l(s)]
                + comm_l[hl, :, pl.ds(s * q, q)]
            )
            ag[0][s] = make_ag(0, s)
            ag[0][s][0].start()
            ag[0][s][1].start()

        for h in range(1, N_DEV - 1):
            for s in range(S):
                prev_r, prev_l = ag[h - 1][s]
                prev_r.wait_recv()
                prev_l.wait_recv()
                ag[h][s] = make_ag(h, s)
                ag[h][s][0].start()
                ag[h][s][1].start()

        for s in range(S):
            ag[N_DEV - 2][s][0].wait_recv()
            ag[N_DEV - 2][s][1].wait_recv()
        for h in range(N_DEV - 1):
            for s in range(S):
                for d in rs[h][s]:
                    d.wait_send()
                for d in ag[h][s]:
                    d.wait_send()

    return pl.pallas_call(
        body,
        out_shape=jax.ShapeDtypeStruct((m, n), jnp.float32),
        in_specs=[
            pl.BlockSpec(memory_space=pltpu.VMEM),
            pl.BlockSpec(memory_space=pltpu.VMEM),
        ],
        out_specs=pl.BlockSpec(memory_space=pltpu.VMEM),
        scratch_shapes=[
            pltpu.VMEM((chunk, n), jnp.float32),
            pltpu.VMEM((N_DEV - 1, chunk, half), jnp.float32),
            pltpu.VMEM((N_DEV - 1, chunk, half), jnp.float32),
            pltpu.SemaphoreType.DMA((N_DEV - 1, S)),
            pltpu.SemaphoreType.DMA((N_DEV - 1, S)),
            pltpu.SemaphoreType.DMA((N_DEV - 1, S)),
            pltpu.SemaphoreType.DMA((N_DEV - 1, S)),
            pltpu.SemaphoreType.DMA((N_DEV - 1, S)),
            pltpu.SemaphoreType.DMA((N_DEV - 1, S)),
            pltpu.SemaphoreType.DMA((N_DEV - 1, S)),
            pltpu.SemaphoreType.DMA((N_DEV - 1, S)),
        ],
        compiler_params=pltpu.CompilerParams(collective_id=0),
    )(x, w_mat)


# device time: 158861 ns/iter; 1.9683x vs baseline; 1.0271x over previous
import jax
import jax.numpy as jnp
from jax import lax
from jax.experimental import pallas as pl
from jax.experimental.pallas import tpu as pltpu

N_DEV = 4
S = 2


def _gelu(y):
    c = 0.7978845608028654
    return 0.5 * y * (1.0 + jnp.tanh(c * (y + 0.044715 * y * y * y)))


def kernel(x, w_mat):
    m, k_per = x.shape
    _, n = w_mat.shape
    chunk = m // N_DEV
    half = n // 2
    q = half // S

    def body(
        x_ref, w_ref, out_ref,
        acc, stage, comm_r, comm_l,
        rs_send_r, rs_recv_r, rs_send_l, rs_recv_l,
        ag_send_r, ag_recv_r, ag_send_l, ag_recv_l,
        cp_r, cp_l,
    ):
        my = lax.axis_index("i")
        left = lax.rem(my + N_DEV - 1, N_DEV)
        right = lax.rem(my + 1, N_DEV)

        barrier = pltpu.get_barrier_semaphore()
        for nbr in (left, right):
            pl.semaphore_signal(
                barrier, inc=1,
                device_id=(nbr,), device_id_type=pl.DeviceIdType.MESH,
            )
        pl.semaphore_wait(barrier, 2)

        def rows(c):
            return pl.ds(c * chunk, chunk)

        def cols_r(s):
            return pl.ds(s * q, q)

        def cols_l(s):
            return pl.ds(half + s * q, q)

        def cs_r(h):
            return lax.rem(my - h + N_DEV, N_DEV)

        def cr_r(h):
            return lax.rem(my - h - 1 + N_DEV, N_DEV)

        def cs_l(h):
            return lax.rem(my + h, N_DEV)

        def cr_l(h):
            return lax.rem(my + h + 1, N_DEV)

        def make_rs(h, s):
            if h == 0:
                src_r = stage.at[pl.ds(0, chunk), cols_r(s)]
                src_l = stage.at[pl.ds(0, chunk), cols_l(s)]
            else:
                src_r = acc.at[rows(cs_r(h)), cols_r(s)]
                src_l = acc.at[rows(cs_l(h)), cols_l(s)]
            rdma_r = pltpu.make_async_remote_copy(
                src_ref=src_r,
                dst_ref=comm_r.at[h, pl.ds(0, chunk), pl.ds(s * q, q)],
                send_sem=rs_send_r.at[h, s],
                recv_sem=rs_recv_r.at[h, s],
                device_id=(right,),
                device_id_type=pl.DeviceIdType.MESH,
            )
            rdma_l = pltpu.make_async_remote_copy(
                src_ref=src_l,
                dst_ref=comm_l.at[h, pl.ds(0, chunk), pl.ds(s * q, q)],
                send_sem=rs_send_l.at[h, s],
                recv_sem=rs_recv_l.at[h, s],
                device_id=(left,),
                device_id_type=pl.DeviceIdType.MESH,
            )
            return rdma_r, rdma_l

        def make_ag(h, s):
            ca = lax.rem(my + 1 - h + N_DEV, N_DEV)
            cb = lax.rem(my - 1 + h + N_DEV, N_DEV)
            rdma_r = pltpu.make_async_remote_copy(
                src_ref=acc.at[rows(ca), cols_r(s)],
                dst_ref=acc.at[rows(ca), cols_r(s)],
                send_sem=ag_send_r.at[h, s],
                recv_sem=ag_recv_r.at[h, s],
                device_id=(right,),
                device_id_type=pl.DeviceIdType.MESH,
            )
            rdma_l = pltpu.make_async_remote_copy(
                src_ref=acc.at[rows(cb), cols_l(s)],
                dst_ref=acc.at[rows(cb), cols_l(s)],
                send_sem=ag_send_l.at[h, s],
                recv_sem=ag_recv_l.at[h, s],
                device_id=(left,),
                device_id_type=pl.DeviceIdType.MESH,
            )
            return rdma_r, rdma_l

        copies = []

        def copy_out(e, s, c_a, c_b):
            cpa = pltpu.make_async_copy(
                acc.at[rows(c_a), cols_r(s)],
                out_ref.at[rows(c_a), cols_r(s)],
                cp_r.at[e, s],
            )
            cpb = pltpu.make_async_copy(
                acc.at[rows(c_b), cols_l(s)],
                out_ref.at[rows(c_b), cols_l(s)],
                cp_l.at[e, s],
            )
            cpa.start()
            cpb.start()
            copies.append(cpa)
            copies.append(cpb)

        stage[:, :] = jnp.dot(
            x_ref[rows(my), :], w_ref[:, :], preferred_element_type=jnp.float32
        )

        rs = [[None] * S for _ in range(N_DEV - 1)]
        ag = [[None] * S for _ in range(N_DEV - 1)]

        for s in range(S):
            rs[0][s] = make_rs(0, s)
            rs[0][s][0].start()
            rs[0][s][1].start()

        for j in range(1, N_DEV):
            c = lax.rem(my + j, N_DEV)
            acc[rows(c), :] = jnp.dot(
                x_ref[rows(c), :], w_ref[:, :],
                preferred_element_type=jnp.float32,
            )

        for h in range(1, N_DEV - 1):
            for s in range(S):
                prev_r, prev_l = rs[h - 1][s]
                prev_r.wait_recv()
                prev_l.wait_recv()
                acc[rows(cr_r(h - 1)), cols_r(s)] = (
                    acc[rows(cr_r(h - 1)), cols_r(s)]
                    + comm_r[h - 1, :, pl.ds(s * q, q)]
                )
                acc[rows(cr_l(h - 1)), cols_l(s)] = (
                    acc[rows(cr_l(h - 1)), cols_l(s)]
                    + comm_l[h - 1, :, pl.ds(s * q, q)]
                )
                rs[h][s] = make_rs(h, s)
                rs[h][s][0].start()
                rs[h][s][1].start()

        hl = N_DEV - 2
        for s in range(S):
            prev_r, prev_l = rs[hl][s]
            prev_r.wait_recv()
            prev_l.wait_recv()
            acc[rows(cr_r(hl)), cols_r(s)] = _gelu(
                acc[rows(cr_r(hl)), cols_r(s)]
                + comm_r[hl, :, pl.ds(s * q, q)]
            )
            acc[rows(cr_l(hl)), cols_l(s)] = _gelu(
                acc[rows(cr_l(hl)), cols_l(s)]
                + comm_l[hl, :, pl.ds(s * q, q)]
            )
            ag[0][s] = make_ag(0, s)
            ag[0][s][0].start()
            ag[0][s][1].start()
            copy_out(0, s, cr_r(hl), cr_l(hl))

        for h in range(1, N_DEV - 1):
            for s in range(S):
                prev_r, prev_l = ag[h - 1][s]
                prev_r.wait_recv()
                prev_l.wait_recv()
                ag[h][s] = make_ag(h, s)
                ag[h][s][0].start()
                ag[h][s][1].start()
                copy_out(
                    h,
                    s,
                    lax.rem(my - (h - 1) + N_DEV, N_DEV),
                    lax.rem(my + (h - 1), N_DEV),
                )

        for s in range(S):
            ag[N_DEV - 2][s][0].wait_recv()
            ag[N_DEV - 2][s][1].wait_recv()
            copy_out(
                N_DEV - 1,
                s,
                lax.rem(my - (N_DEV - 2) + N_DEV, N_DEV),
                lax.rem(my + (N_DEV - 2), N_DEV),
            )
        for cp in copies:
            cp.wait()
        for h in range(N_DEV - 1):
            for s in range(S):
                for d in rs[h][s]:
                    d.wait_send()
                for d in ag[h][s]:
                    d.wait_send()

    return pl.pallas_call(
        body,
        out_shape=jax.ShapeDtypeStruct((m, n), jnp.float32),
        in_specs=[
            pl.BlockSpec(memory_space=pltpu.VMEM),
            pl.BlockSpec(memory_space=pltpu.VMEM),
        ],
        out_specs=pl.BlockSpec(memory_space=pl.ANY),
        scratch_shapes=[
            pltpu.VMEM((m, n), jnp.float32),
            pltpu.VMEM((chunk, n), jnp.float32),
            pltpu.VMEM((N_DEV - 1, chunk, half), jnp.float32),
            pltpu.VMEM((N_DEV - 1, chunk, half), jnp.float32),
            pltpu.SemaphoreType.DMA((N_DEV - 1, S)),
            pltpu.SemaphoreType.DMA((N_DEV - 1, S)),
            pltpu.SemaphoreType.DMA((N_DEV - 1, S)),
            pltpu.SemaphoreType.DMA((N_DEV - 1, S)),
            pltpu.SemaphoreType.DMA((N_DEV - 1, S)),
            pltpu.SemaphoreType.DMA((N_DEV - 1, S)),
            pltpu.SemaphoreType.DMA((N_DEV - 1, S)),
            pltpu.SemaphoreType.DMA((N_DEV - 1, S)),
            pltpu.SemaphoreType.DMA((N_DEV, S)),
            pltpu.SemaphoreType.DMA((N_DEV, S)),
        ],
        compiler_params=pltpu.CompilerParams(collective_id=0),
    )(x, w_mat)


# device time: 156716 ns/iter; 1.9952x vs baseline; 1.0137x over previous
import jax
import jax.numpy as jnp
from jax import lax
from jax.experimental import pallas as pl
from jax.experimental.pallas import tpu as pltpu

N_DEV = 4
S = 2


def _gelu(y):
    c = 0.7978845608028654
    return 0.5 * y * (1.0 + jnp.tanh(c * (y + 0.044715 * y * y * y)))


def kernel(x, w_mat):
    m, k_per = x.shape
    _, n = w_mat.shape
    chunk = m // N_DEV
    half = n // 2
    q = half // S

    def body(
        x_hbm, w_hbm, out_ref,
        x_v, w_v, acc, stage, comm_r, comm_l,
        in_sems,
        rs_send_r, rs_recv_r, rs_send_l, rs_recv_l,
        ag_send_r, ag_recv_r, ag_send_l, ag_recv_l,
        cp_r, cp_l,
    ):
        my = lax.axis_index("i")
        left = lax.rem(my + N_DEV - 1, N_DEV)
        right = lax.rem(my + 1, N_DEV)

        def rows(c):
            return pl.ds(c * chunk, chunk)

        def cols_r(s):
            return pl.ds(s * q, q)

        def cols_l(s):
            return pl.ds(half + s * q, q)

        in_cps = []
        for i, (src, dst) in enumerate([
            (w_hbm.at[:, cols_r(0)], w_v.at[:, cols_r(0)]),
            (w_hbm.at[:, cols_l(0)], w_v.at[:, cols_l(0)]),
            (x_hbm.at[:, :], x_v.at[:, :]),
            (w_hbm.at[:, cols_r(1)], w_v.at[:, cols_r(1)]),
            (w_hbm.at[:, cols_l(1)], w_v.at[:, cols_l(1)]),
        ]):
            cp = pltpu.make_async_copy(src, dst, in_sems.at[i])
            cp.start()
            in_cps.append(cp)

        barrier = pltpu.get_barrier_semaphore()
        for nbr in (left, right):
            pl.semaphore_signal(
                barrier, inc=1,
                device_id=(nbr,), device_id_type=pl.DeviceIdType.MESH,
            )
        pl.semaphore_wait(barrier, 2)

        def cs_r(h):
            return lax.rem(my - h + N_DEV, N_DEV)

        def cr_r(h):
            return lax.rem(my - h - 1 + N_DEV, N_DEV)

        def cs_l(h):
            return lax.rem(my + h, N_DEV)

        def cr_l(h):
            return lax.rem(my + h + 1, N_DEV)

        def make_rs_r(h, s):
            src = (
                stage.at[pl.ds(0, chunk), cols_r(s)]
                if h == 0
                else acc.at[rows(cs_r(h)), cols_r(s)]
            )
            return pltpu.make_async_remote_copy(
                src_ref=src,
                dst_ref=comm_r.at[h, pl.ds(0, chunk), pl.ds(s * q, q)],
                send_sem=rs_send_r.at[h, s],
                recv_sem=rs_recv_r.at[h, s],
                device_id=(right,),
                device_id_type=pl.DeviceIdType.MESH,
            )

        def make_rs_l(h, s):
            src = (
                stage.at[pl.ds(0, chunk), cols_l(s)]
                if h == 0
                else acc.at[rows(cs_l(h)), cols_l(s)]
            )
            return pltpu.make_async_remote_copy(
                src_ref=src,
                dst_ref=comm_l.at[h, pl.ds(0, chunk), pl.ds(s * q, q)],
                send_sem=rs_send_l.at[h, s],
                recv_sem=rs_recv_l.at[h, s],
                device_id=(left,),
                device_id_type=pl.DeviceIdType.MESH,
            )

        def make_ag_r(h, s):
            ca = lax.rem(my + 1 - h + N_DEV, N_DEV)
            return pltpu.make_async_remote_copy(
                src_ref=acc.at[rows(ca), cols_r(s)],
                dst_ref=acc.at[rows(ca), cols_r(s)],
                send_sem=ag_send_r.at[h, s],
                recv_sem=ag_recv_r.at[h, s],
                device_id=(right,),
                device_id_type=pl.DeviceIdType.MESH,
            )

        def make_ag_l(h, s):
            cb = lax.rem(my - 1 + h + N_DEV, N_DEV)
            return pltpu.make_async_remote_copy(
                src_ref=acc.at[rows(cb), cols_l(s)],
                dst_ref=acc.at[rows(cb), cols_l(s)],
                send_sem=ag_send_l.at[h, s],
                recv_sem=ag_recv_l.at[h, s],
                device_id=(left,),
                device_id_type=pl.DeviceIdType.MESH,
            )

        copies = []

        def copy_out_r(e, s, c_a):
            cp = pltpu.make_async_copy(
                acc.at[rows(c_a), cols_r(s)],
                out_ref.at[rows(c_a), cols_r(s)],
                cp_r.at[e, s],
            )
            cp.start()
            copies.append(cp)

        def copy_out_l(e, s, c_b):
            cp = pltpu.make_async_copy(
                acc.at[rows(c_b), cols_l(s)],
                out_ref.at[rows(c_b), cols_l(s)],
                cp_l.at[e, s],
            )
            cp.start()
            copies.append(cp)

        rs = [[[None, None] for _ in range(S)] for _ in range(N_DEV - 1)]
        ag = [[[None, None] for _ in range(S)] for _ in range(N_DEV - 1)]

        in_cps[0].wait()
        in_cps[1].wait()
        in_cps[2].wait()
        stage[pl.ds(0, chunk), cols_r(0)] = jnp.dot(
            x_v[rows(my), :], w_v[:, cols_r(0)],
            preferred_element_type=jnp.float32,
        )
        stage[pl.ds(0, chunk), cols_l(0)] = jnp.dot(
            x_v[rows(my), :], w_v[:, cols_l(0)],
            preferred_element_type=jnp.float32,
        )
        rs[0][0][0] = make_rs_r(0, 0)
        rs[0][0][1] = make_rs_l(0, 0)
        rs[0][0][0].start()
        rs[0][0][1].start()

        in_cps[3].wait()
        in_cps[4].wait()
        stage[pl.ds(0, chunk), cols_r(1)] = jnp.dot(
            x_v[rows(my), :], w_v[:, cols_r(1)],
            preferred_element_type=jnp.float32,
        )
        stage[pl.ds(0, chunk), cols_l(1)] = jnp.dot(
            x_v[rows(my), :], w_v[:, cols_l(1)],
            preferred_element_type=jnp.float32,
        )
        rs[0][1][0] = make_rs_r(0, 1)
        rs[0][1][1] = make_rs_l(0, 1)
        rs[0][1][0].start()
        rs[0][1][1].start()

        for j in range(1, N_DEV):
            c = lax.rem(my + j, N_DEV)
            acc[rows(c), :] = jnp.dot(
                x_v[rows(c), :], w_v[:, :],
                preferred_element_type=jnp.float32,
            )

        for h in range(1, N_DEV - 1):
            for s in range(S):
                rs[h - 1][s][0].wait_recv()
                acc[rows(cr_r(h - 1)), cols_r(s)] = (
                    acc[rows(cr_r(h - 1)), cols_r(s)]
                    + comm_r[h - 1, :, pl.ds(s * q, q)]
                )
                rs[h][s][0] = make_rs_r(h, s)
                rs[h][s][0].start()

                rs[h - 1][s][1].wait_recv()
                acc[rows(cr_l(h - 1)), cols_l(s)] = (
                    acc[rows(cr_l(h - 1)), cols_l(s)]
                    + comm_l[h - 1, :, pl.ds(s * q, q)]
                )
                rs[h][s][1] = make_rs_l(h, s)
                rs[h][s][1].start()

        hl = N_DEV - 2
        for s in range(S):
            rs[hl][s][0].wait_recv()
            acc[rows(cr_r(hl)), cols_r(s)] = _gelu(
                acc[rows(cr_r(hl)), cols_r(s)]
                + comm_r[hl, :, pl.ds(s * q, q)]
            )
            ag[0][s][0] = make_ag_r(0, s)
            ag[0][s][0].start()
            copy_out_r(0, s, cr_r(hl))

            rs[hl][s][1].wait_recv()
            acc[rows(cr_l(hl)), cols_l(s)] = _gelu(
                acc[rows(cr_l(hl)), cols_l(s)]
                + comm_l[hl, :, pl.ds(s * q, q)]
            )
            ag[0][s][1] = make_ag_l(0, s)
            ag[0][s][1].start()
            copy_out_l(0, s, cr_l(hl))

        for h in range(1, N_DEV - 1):
            for s in range(S):
                ag[h - 1][s][0].wait_recv()
                ag[h][s][0] = make_ag_r(h, s)
                ag[h][s][0].start()
                copy_out_r(h, s, lax.rem(my - (h - 1) + N_DEV, N_DEV))

                ag[h - 1][s][1].wait_recv()
                ag[h][s][1] = make_ag_l(h, s)
                ag[h][s][1].start()
                copy_out_l(h, s, lax.rem(my + (h - 1), N_DEV))

        for s in range(S):
            ag[N_DEV - 2][s][0].wait_recv()
            copy_out_r(N_DEV - 1, s, lax.rem(my - (N_DEV - 2) + N_DEV, N_DEV))
            ag[N_DEV - 2][s][1].wait_recv()
            copy_out_l(N_DEV - 1, s, lax.rem(my + (N_DEV - 2), N_DEV))
        for cp in copies:
            cp.wait()
        for h in range(N_DEV - 1):
            for s in range(S):
                rs[h][s][0].wait_send()
                rs[h][s][1].wait_send()
                ag[h][s][0].wait_send()
                ag[h][s][1].wait_send()

    return pl.pallas_call(
        body,
        out_shape=jax.ShapeDtypeStruct((m, n), jnp.float32),
        in_specs=[
            pl.BlockSpec(memory_space=pl.ANY),
            pl.BlockSpec(memory_space=pl.ANY),
        ],
        out_specs=pl.BlockSpec(memory_space=pl.ANY),
        scratch_shapes=[
            pltpu.VMEM((m, k_per), jnp.float32),
            pltpu.VMEM((k_per, n), jnp.float32),
            pltpu.VMEM((m, n), jnp.float32),
            pltpu.VMEM((chunk, n), jnp.float32),
            pltpu.VMEM((N_DEV - 1, chunk, half), jnp.float32),
            pltpu.VMEM((N_DEV - 1, chunk, half), jnp.float32),
            pltpu.SemaphoreType.DMA((5,)),
            pltpu.SemaphoreType.DMA((N_DEV - 1, S)),
            pltpu.SemaphoreType.DMA((N_DEV - 1, S)),
            pltpu.SemaphoreType.DMA((N_DEV - 1, S)),
            pltpu.SemaphoreType.DMA((N_DEV - 1, S)),
            pltpu.SemaphoreType.DMA((N_DEV - 1, S)),
            pltpu.SemaphoreType.DMA((N_DEV - 1, S)),
            pltpu.SemaphoreType.DMA((N_DEV - 1, S)),
            pltpu.SemaphoreType.DMA((N_DEV - 1, S)),
            pltpu.SemaphoreType.DMA((N_DEV, S)),
            pltpu.SemaphoreType.DMA((N_DEV, S)),
        ],
        compiler_params=pltpu.CompilerParams(
            collective_id=0,
            vmem_limit_bytes=64 * 1024 * 1024,
        ),
    )(x, w_mat)


# device time: 89408 ns/iter; 3.4973x vs baseline; 1.7528x over previous
import jax
import jax.numpy as jnp
from jax import lax
from jax.experimental import pallas as pl
from jax.experimental.pallas import tpu as pltpu

N_DEV = 4
S = 2


def _gelu(y):
    c = 0.7978845608028654
    return 0.5 * y * (1.0 + jnp.tanh(c * (y + 0.044715 * y * y * y)))


def kernel(x, w_mat):
    m, k_per = x.shape
    _, n = w_mat.shape
    chunk = m // N_DEV
    half = n // 2
    q = half // S

    def body(
        x_hbm, w_hbm, out_ref,
        x_v, w_v, acc, stage16, gath16, comm_r, comm_l, sbuf_r, sbuf_l,
        in_sems,
        rs_send_r, rs_recv_r, rs_send_l, rs_recv_l,
        ag_send_r, ag_recv_r, ag_send_l, ag_recv_l,
        cp_r, cp_l,
    ):
        my = lax.axis_index("i")
        left = lax.rem(my + N_DEV - 1, N_DEV)
        right = lax.rem(my + 1, N_DEV)

        def rows(c):
            return pl.ds(c * chunk, chunk)

        def cols_r(s):
            return pl.ds(s * q, q)

        def cols_l(s):
            return pl.ds(half + s * q, q)

        def sub(s):
            return pl.ds(s * q, q)

        in_cps = []
        for i, (src, dst) in enumerate([
            (w_hbm.at[:, cols_r(0)], w_v.at[:, cols_r(0)]),
            (w_hbm.at[:, cols_l(0)], w_v.at[:, cols_l(0)]),
            (x_hbm.at[:, :], x_v.at[:, :]),
            (w_hbm.at[:, cols_r(1)], w_v.at[:, cols_r(1)]),
            (w_hbm.at[:, cols_l(1)], w_v.at[:, cols_l(1)]),
        ]):
            cp = pltpu.make_async_copy(src, dst, in_sems.at[i])
            cp.start()
            in_cps.append(cp)

        barrier = pltpu.get_barrier_semaphore()
        for nbr in (left, right):
            pl.semaphore_signal(
                barrier, inc=1,
                device_id=(nbr,), device_id_type=pl.DeviceIdType.MESH,
            )
        pl.semaphore_wait(barrier, 2)

        def cr_r(h):
            return lax.rem(my - h - 1 + N_DEV, N_DEV)

        def cr_l(h):
            return lax.rem(my + h + 1, N_DEV)

        def make_rs_r(h, s):
            src = (
                stage16.at[pl.ds(0, chunk), cols_r(s)]
                if h == 0
                else sbuf_r.at[h - 1, pl.ds(0, chunk), sub(s)]
            )
            return pltpu.make_async_remote_copy(
                src_ref=src,
                dst_ref=comm_r.at[h, pl.ds(0, chunk), sub(s)],
                send_sem=rs_send_r.at[h, s],
                recv_sem=rs_recv_r.at[h, s],
                device_id=(right,),
                device_id_type=pl.DeviceIdType.MESH,
            )

        def make_rs_l(h, s):
            src = (
                stage16.at[pl.ds(0, chunk), cols_l(s)]
                if h == 0
                else sbuf_l.at[h - 1, pl.ds(0, chunk), sub(s)]
            )
            return pltpu.make_async_remote_copy(
                src_ref=src,
                dst_ref=comm_l.at[h, pl.ds(0, chunk), sub(s)],
                send_sem=rs_send_l.at[h, s],
                recv_sem=rs_recv_l.at[h, s],
                device_id=(left,),
                device_id_type=pl.DeviceIdType.MESH,
            )

        def make_ag_r(h, s):
            ca = lax.rem(my + 1 - h + N_DEV, N_DEV)
            return pltpu.make_async_remote_copy(
                src_ref=gath16.at[rows(ca), cols_r(s)],
                dst_ref=gath16.at[rows(ca), cols_r(s)],
                send_sem=ag_send_r.at[h, s],
                recv_sem=ag_recv_r.at[h, s],
                device_id=(right,),
                device_id_type=pl.DeviceIdType.MESH,
            )

        def make_ag_l(h, s):
            cb = lax.rem(my - 1 + h + N_DEV, N_DEV)
            return pltpu.make_async_remote_copy(
                src_ref=gath16.at[rows(cb), cols_l(s)],
                dst_ref=gath16.at[rows(cb), cols_l(s)],
                send_sem=ag_send_l.at[h, s],
                recv_sem=ag_recv_l.at[h, s],
                device_id=(left,),
                device_id_type=pl.DeviceIdType.MESH,
            )

        copies = []

        def copy_out_r(e, s, c_a):
            cp = pltpu.make_async_copy(
                acc.at[rows(c_a), cols_r(s)],
                out_ref.at[rows(c_a), cols_r(s)],
                cp_r.at[e, s],
            )
            cp.start()
            copies.append(cp)

        def copy_out_l(e, s, c_b):
            cp = pltpu.make_async_copy(
                acc.at[rows(c_b), cols_l(s)],
                out_ref.at[rows(c_b), cols_l(s)],
                cp_l.at[e, s],
            )
            cp.start()
            copies.append(cp)

        rs = [[[None, None] for _ in range(S)] for _ in range(N_DEV - 1)]
        ag = [[[None, None] for _ in range(S)] for _ in range(N_DEV - 1)]

        in_cps[0].wait()
        in_cps[1].wait()
        in_cps[2].wait()
        stage16[pl.ds(0, chunk), cols_r(0)] = jnp.dot(
            x_v[rows(my), :], w_v[:, cols_r(0)],
            preferred_element_type=jnp.float32,
        ).astype(jnp.bfloat16)
        stage16[pl.ds(0, chunk), cols_l(0)] = jnp.dot(
            x_v[rows(my), :], w_v[:, cols_l(0)],
            preferred_element_type=jnp.float32,
        ).astype(jnp.bfloat16)
        rs[0][0][0] = make_rs_r(0, 0)
        rs[0][0][1] = make_rs_l(0, 0)
        rs[0][0][0].start()
        rs[0][0][1].start()

        in_cps[3].wait()
        in_cps[4].wait()
        stage16[pl.ds(0, chunk), cols_r(1)] = jnp.dot(
            x_v[rows(my), :], w_v[:, cols_r(1)],
            preferred_element_type=jnp.float32,
        ).astype(jnp.bfloat16)
        stage16[pl.ds(0, chunk), cols_l(1)] = jnp.dot(
            x_v[rows(my), :], w_v[:, cols_l(1)],
            preferred_element_type=jnp.float32,
        ).astype(jnp.bfloat16)
        rs[0][1][0] = make_rs_r(0, 1)
        rs[0][1][1] = make_rs_l(0, 1)
        rs[0][1][0].start()
        rs[0][1][1].start()

        for j in (1, N_DEV - 1, 2):
            c = lax.rem(my + j, N_DEV)
            acc[rows(c), :] = jnp.dot(
                x_v[rows(c), :], w_v[:, :],
                preferred_element_type=jnp.float32,
            )

        for h in range(1, N_DEV - 1):
            for s in range(S):
                rs[h - 1][s][0].wait_recv()
                sbuf_r[h - 1, pl.ds(0, chunk), sub(s)] = (
                    acc[rows(cr_r(h - 1)), cols_r(s)]
                    + comm_r[h - 1, :, sub(s)].astype(jnp.float32)
                ).astype(jnp.bfloat16)
                rs[h][s][0] = make_rs_r(h, s)
                rs[h][s][0].start()

                rs[h - 1][s][1].wait_recv()
                sbuf_l[h - 1, pl.ds(0, chunk), sub(s)] = (
                    acc[rows(cr_l(h - 1)), cols_l(s)]
                    + comm_l[h - 1, :, sub(s)].astype(jnp.float32)
                ).astype(jnp.bfloat16)
                rs[h][s][1] = make_rs_l(h, s)
                rs[h][s][1].start()

        hl = N_DEV - 2
        for s in range(S):
            rs[hl][s][0].wait_recv()
            g = _gelu(
                acc[rows(cr_r(hl)), cols_r(s)]
                + comm_r[hl, :, sub(s)].astype(jnp.float32)
            )
            acc[rows(cr_r(hl)), cols_r(s)] = g
            gath16[rows(cr_r(hl)), cols_r(s)] = g.astype(jnp.bfloat16)
            ag[0][s][0] = make_ag_r(0, s)
            ag[0][s][0].start()
            copy_out_r(0, s, cr_r(hl))

            rs[hl][s][1].wait_recv()
            g = _gelu(
                acc[rows(cr_l(hl)), cols_l(s)]
                + comm_l[hl, :, sub(s)].astype(jnp.float32)
            )
            acc[rows(cr_l(hl)), cols_l(s)] = g
            gath16[rows(cr_l(hl)), cols_l(s)] = g.astype(jnp.bfloat16)
            ag[0][s][1] = make_ag_l(0, s)
            ag[0][s][1].start()
            copy_out_l(0, s, cr_l(hl))

        for h in range(1, N_DEV - 1):
            for s in range(S):
                ca = lax.rem(my - (h - 1) + N_DEV, N_DEV)
                ag[h - 1][s][0].wait_recv()
                ag[h][s][0] = make_ag_r(h, s)
                ag[h][s][0].start()
                acc[rows(ca), cols_r(s)] = (
                    gath16[rows(ca), cols_r(s)].astype(jnp.float32)
                )
                copy_out_r(h, s, ca)

                cb = lax.rem(my + (h - 1), N_DEV)
                ag[h - 1][s][1].wait_recv()
                ag[h][s][1] = make_ag_l(h, s)
                ag[h][s][1].start()
                acc[rows(cb), cols_l(s)] = (
                    gath16[rows(cb), cols_l(s)].astype(jnp.float32)
                )
                copy_out_l(h, s, cb)

        for s in range(S):
            ca = lax.rem(my - (N_DEV - 2) + N_DEV, N_DEV)
            ag[N_DEV - 2][s][0].wait_recv()
            acc[rows(ca), cols_r(s)] = (
                gath16[rows(ca), cols_r(s)].astype(jnp.float32)
            )
            copy_out_r(N_DEV - 1, s, ca)

            cb = lax.rem(my + (N_DEV - 2), N_DEV)
            ag[N_DEV - 2][s][1].wait_recv()
            acc[rows(cb), cols_l(s)] = (
                gath16[rows(cb), cols_l(s)].astype(jnp.float32)
            )
            copy_out_l(N_DEV - 1, s, cb)
        for cp in copies:
            cp.wait()
        for h in range(N_DEV - 1):
            for s in range(S):
                rs[h][s][0].wait_send()
                rs[h][s][1].wait_send()
                ag[h][s][0].wait_send()
                ag[h][s][1].wait_send()

    return pl.pallas_call(
        body,
        out_shape=jax.ShapeDtypeStruct((m, n), jnp.float32),
        in_specs=[
            pl.BlockSpec(memory_space=pl.ANY),
            pl.BlockSpec(memory_space=pl.ANY),
        ],
        out_specs=pl.BlockSpec(memory_space=pl.ANY),
        scratch_shapes=[
            pltpu.VMEM((m, k_per), jnp.float32),
            pltpu.VMEM((k_per, n), jnp.float32),
            pltpu.VMEM((m, n), jnp.float32),
            pltpu.VMEM((chunk, n), jnp.bfloat16),
            pltpu.VMEM((m, n), jnp.bfloat16),
            pltpu.VMEM((N_DEV - 1, chunk, half), jnp.bfloat16),
            pltpu.VMEM((N_DEV - 1, chunk, half), jnp.bfloat16),
            pltpu.VMEM((N_DEV - 2, chunk, half), jnp.bfloat16),
            pltpu.VMEM((N_DEV - 2, chunk, half), jnp.bfloat16),
            pltpu.SemaphoreType.DMA((5,)),
            pltpu.SemaphoreType.DMA((N_DEV - 1, S)),
            pltpu.SemaphoreType.DMA((N_DEV - 1, S)),
            pltpu.SemaphoreType.DMA((N_DEV - 1, S)),
            pltpu.SemaphoreType.DMA((N_DEV - 1, S)),
            pltpu.SemaphoreType.DMA((N_DEV - 1, S)),
            pltpu.SemaphoreType.DMA((N_DEV - 1, S)),
            pltpu.SemaphoreType.DMA((N_DEV - 1, S)),
            pltpu.SemaphoreType.DMA((N_DEV - 1, S)),
            pltpu.SemaphoreType.DMA((N_DEV, S)),
            pltpu.SemaphoreType.DMA((N_DEV, S)),
        ],
        compiler_params=pltpu.CompilerParams(
            collective_id=0,
            vmem_limit_bytes=64 * 1024 * 1024,
        ),
    )(x, w_mat)
